# initial kernel scaffold (unmeasured)
import jax
import jax.numpy as jnp
from jax import lax
from jax.experimental import pallas as pl
from jax.experimental.pallas import tpu as pltpu

B, SQ, H, D = 4, 32, 8, 128
BH = B * H
SCALE = D ** -0.5
CHUNK = 512
NEG_INF = -1e30


def kernel(Q, K, V):
    b, sq, h, d = Q.shape
    skv = K.shape[1]
    assert (b, sq, h, d) == (B, SQ, H, D), Q.shape
    assert skv % CHUNK == 0, skv
    nc = skv // CHUNK

    def body(q_ref, k_ref, v_ref, o_ref,
             acc_ref, m_ref, l_ref,
             z_send, z_recv, l_snd, l_rcv,
             send_sems, recv_sems):
        step = pl.program_id(0)
        my_x = lax.axis_index("x")
        my_y = lax.axis_index("y")
        nbr = (1 - my_x, my_y)

        @pl.when(step == 0)
        def _init():
            bsem = pltpu.get_barrier_semaphore()
            pl.semaphore_signal(bsem, inc=1, device_id=nbr,
                                device_id_type=pl.DeviceIdType.MESH)
            pl.semaphore_wait(bsem, 1)
            acc_ref[...] = jnp.zeros_like(acc_ref)
            m_ref[...] = jnp.full_like(m_ref, NEG_INF)
            l_ref[...] = jnp.zeros_like(l_ref)

        for bi in range(B):
            for hi in range(H):
                col = bi * H + hi
                q = q_ref[bi, :, hi, :].astype(jnp.bfloat16)
                k = k_ref[bi, :, hi, :].astype(jnp.bfloat16)
                v = v_ref[bi, :, hi, :].astype(jnp.bfloat16)
                s = lax.dot_general(
                    q, k, (((1,), (1,)), ((), ())),
                    preferred_element_type=jnp.float32) * SCALE
                m_prev = m_ref[:, col:col + 1]
                l_prev = l_ref[:, col:col + 1]
                m_cur = jnp.maximum(m_prev, jnp.max(s, axis=1, keepdims=True))
                p = jnp.exp(s - m_cur)
                corr = jnp.exp(m_prev - m_cur)
                l_new = l_prev * corr + jnp.sum(p, axis=1, keepdims=True)
                pv = lax.dot_general(
                    p.astype(jnp.bfloat16), v, (((1,), (0,)), ((), ())),
                    preferred_element_type=jnp.float32)
                acc_ref[bi, hi] = acc_ref[bi, hi] * corr + pv
                m_ref[:, col:col + 1] = m_cur
                l_ref[:, col:col + 1] = l_new

        @pl.when(step == nc - 1)
        def _finish():
            for bi in range(B):
                for hi in range(H):
                    col = bi * H + hi
                    z_send[bi, hi] = (
                        acc_ref[bi, hi] / l_ref[:, col:col + 1]
                    ).astype(jnp.bfloat16)
            l_snd[...] = m_ref[...] + jnp.log(l_ref[...])

            rdma_z = pltpu.make_async_remote_copy(
                src_ref=z_send, dst_ref=z_recv,
                send_sem=send_sems.at[0], recv_sem=recv_sems.at[0],
                device_id=nbr, device_id_type=pl.DeviceIdType.MESH)
            rdma_l = pltpu.make_async_remote_copy(
                src_ref=l_snd, dst_ref=l_rcv,
                send_sem=send_sems.at[1], recv_sem=recv_sems.at[1],
                device_id=nbr, device_id_type=pl.DeviceIdType.MESH)
            rdma_z.start()
            rdma_l.start()
            rdma_z.wait()
            rdma_l.wait()

            for bi in range(B):
                for hi in range(H):
                    col = bi * H + hi
                    L_s = l_snd[:, col:col + 1]
                    L_o = l_rcv[:, col:col + 1]
                    m_t = jnp.maximum(L_s, L_o)
                    w_s = jnp.exp(L_s - m_t)
                    w_o = jnp.exp(L_o - m_t)
                    z_s = acc_ref[bi, hi] / l_ref[:, col:col + 1]
                    z_o = z_recv[bi, hi].astype(jnp.float32)
                    o_ref[bi, :, hi, :] = (z_s * w_s + z_o * w_o) / (w_s + w_o)

    return pl.pallas_call(
        body,
        grid=(nc,),
        in_specs=[
            pl.BlockSpec((B, SQ, H, D), lambda i: (0, 0, 0, 0)),
            pl.BlockSpec((B, CHUNK, H, D), lambda i: (0, i, 0, 0)),
            pl.BlockSpec((B, CHUNK, H, D), lambda i: (0, i, 0, 0)),
        ],
        out_specs=pl.BlockSpec((B, SQ, H, D), lambda i: (0, 0, 0, 0)),
        out_shape=jax.ShapeDtypeStruct((B, SQ, H, D), jnp.float32),
        scratch_shapes=[
            pltpu.VMEM((B, H, SQ, D), jnp.float32),
            pltpu.VMEM((SQ, BH), jnp.float32),
            pltpu.VMEM((SQ, BH), jnp.float32),
            pltpu.VMEM((B, H, SQ, D), jnp.bfloat16),
            pltpu.VMEM((B, H, SQ, D), jnp.bfloat16),
            pltpu.VMEM((SQ, BH), jnp.float32),
            pltpu.VMEM((SQ, BH), jnp.float32),
            pltpu.SemaphoreType.DMA((2,)),
            pltpu.SemaphoreType.DMA((2,)),
        ],
        compiler_params=pltpu.CompilerParams(
            dimension_semantics=("arbitrary",),
            collective_id=0,
        ),
    )(Q, K, V)


# baseline (device time: 234635 ns/iter reference)
import jax
import jax.numpy as jnp
from jax import lax
from jax.experimental import pallas as pl
from jax.experimental.pallas import tpu as pltpu

B, SQ, H, D = 4, 32, 8, 128
BH = B * H
SCALE = D ** -0.5
CHUNK = 256
NEG_INF = -1e30


def kernel(Q, K, V):
    b, sq, h, d = Q.shape
    skv = K.shape[1]
    assert (b, sq, h, d) == (B, SQ, H, D), Q.shape
    assert skv % CHUNK == 0, skv
    nc = skv // CHUNK

    def body(q_ref, k_ref, v_ref, o_ref,
             acc_ref, m_ref, l_ref,
             z_send, z_recv, l_snd, l_rcv,
             send_sems, recv_sems):
        step = pl.program_id(0)
        my_x = lax.axis_index("x")
        my_y = lax.axis_index("y")
        nbr = (1 - my_x, my_y)

        @pl.when(step == 0)
        def _init():
            bsem = pltpu.get_barrier_semaphore()
            pl.semaphore_signal(bsem, inc=1, device_id=nbr,
                                device_id_type=pl.DeviceIdType.MESH)
            pl.semaphore_wait(bsem, 1)
            acc_ref[...] = jnp.zeros_like(acc_ref)
            m_ref[...] = jnp.full_like(m_ref, NEG_INF)
            l_ref[...] = jnp.zeros_like(l_ref)

        for bi in range(B):
            for hi in range(H):
                col = bi * H + hi
                q = q_ref[bi, :, hi, :].astype(jnp.bfloat16)
                k = k_ref[bi, :, hi, :].astype(jnp.bfloat16)
                v = v_ref[bi, :, hi, :].astype(jnp.bfloat16)
                s = lax.dot_general(
                    q, k, (((1,), (1,)), ((), ())),
                    preferred_element_type=jnp.float32) * SCALE
                m_prev = m_ref[:, col:col + 1]
                l_prev = l_ref[:, col:col + 1]
                m_cur = jnp.maximum(m_prev, jnp.max(s, axis=1, keepdims=True))
                p = jnp.exp(s - m_cur)
                corr = jnp.exp(m_prev - m_cur)
                l_new = l_prev * corr + jnp.sum(p, axis=1, keepdims=True)
                pv = lax.dot_general(
                    p.astype(jnp.bfloat16), v, (((1,), (0,)), ((), ())),
                    preferred_element_type=jnp.float32)
                acc_ref[bi, hi] = acc_ref[bi, hi] * corr + pv
                m_ref[:, col:col + 1] = m_cur
                l_ref[:, col:col + 1] = l_new

        @pl.when(step == nc - 1)
        def _finish():
            for bi in range(B):
                for hi in range(H):
                    col = bi * H + hi
                    z_send[bi, hi] = (
                        acc_ref[bi, hi] / l_ref[:, col:col + 1]
                    ).astype(jnp.bfloat16)
            l_snd[...] = m_ref[...] + jnp.log(l_ref[...])

            rdma_z = pltpu.make_async_remote_copy(
                src_ref=z_send, dst_ref=z_recv,
                send_sem=send_sems.at[0], recv_sem=recv_sems.at[0],
                device_id=nbr, device_id_type=pl.DeviceIdType.MESH)
            rdma_l = pltpu.make_async_remote_copy(
                src_ref=l_snd, dst_ref=l_rcv,
                send_sem=send_sems.at[1], recv_sem=recv_sems.at[1],
                device_id=nbr, device_id_type=pl.DeviceIdType.MESH)
            rdma_z.start()
            rdma_l.start()
            rdma_z.wait()
            rdma_l.wait()

            for bi in range(B):
                for hi in range(H):
                    col = bi * H + hi
                    L_s = l_snd[:, col:col + 1]
                    L_o = l_rcv[:, col:col + 1]
                    m_t = jnp.maximum(L_s, L_o)
                    w_s = jnp.exp(L_s - m_t)
                    w_o = jnp.exp(L_o - m_t)
                    z_s = acc_ref[bi, hi] / l_ref[:, col:col + 1]
                    z_o = z_recv[bi, hi].astype(jnp.float32)
                    o_ref[bi, :, hi, :] = (z_s * w_s + z_o * w_o) / (w_s + w_o)

    return pl.pallas_call(
        body,
        grid=(nc,),
        in_specs=[
            pl.BlockSpec((B, SQ, H, D), lambda i: (0, 0, 0, 0)),
            pl.BlockSpec((B, CHUNK, H, D), lambda i: (0, i, 0, 0)),
            pl.BlockSpec((B, CHUNK, H, D), lambda i: (0, i, 0, 0)),
        ],
        out_specs=pl.BlockSpec((B, SQ, H, D), lambda i: (0, 0, 0, 0)),
        out_shape=jax.ShapeDtypeStruct((B, SQ, H, D), jnp.float32),
        scratch_shapes=[
            pltpu.VMEM((B, H, SQ, D), jnp.float32),
            pltpu.VMEM((SQ, BH), jnp.float32),
            pltpu.VMEM((SQ, BH), jnp.float32),
            pltpu.VMEM((B, H, SQ, D), jnp.bfloat16),
            pltpu.VMEM((B, H, SQ, D), jnp.bfloat16),
            pltpu.VMEM((SQ, BH), jnp.float32),
            pltpu.VMEM((SQ, BH), jnp.float32),
            pltpu.SemaphoreType.DMA((2,)),
            pltpu.SemaphoreType.DMA((2,)),
        ],
        compiler_params=pltpu.CompilerParams(
            dimension_semantics=("arbitrary",),
            collective_id=0,
        ),
    )(Q, K, V)
